# baseline (speedup 1.0000x reference)
import jax
import jax.numpy as jnp
from jax.experimental import pallas as pl
from jax.experimental.pallas import tpu as pltpu

_SLOPE = 0.3
_EPS = 1e-5


def _leaky(v):
    return jnp.where(v >= 0, v, _SLOPE * v)


def _block_kernel(x_ref, w1_ref, t1_ref, w2_ref, t2_ref, o_ref, xw_ref, yw_ref):
    D, H, W, C = x_ref.shape
    NQ = W // 2
    KW = 4 * C
    NC = 2 * C
    M = D * H * NQ
    bf16 = jnp.bfloat16

    zd = jnp.zeros((1, H + 2, NQ, KW), bf16)
    zh = jnp.zeros((D, 1, NQ, KW), bf16)
    zc = jnp.zeros((D, H, 1, C), bf16)
    for ref in (xw_ref, yw_ref):
        ref[0:1, :, :, :] = zd
        ref[D + 1:D + 2, :, :, :] = zd
        ref[1:1 + D, 0:1, :, :] = zh
        ref[1:1 + D, H + 1:H + 2, :, :] = zh
        ref[1:1 + D, 1:1 + H, 0:1, 0:C] = zc
        ref[1:1 + D, 1:1 + H, NQ - 1:NQ, KW - C:KW] = zc

    xb = x_ref[...].astype(bf16)
    for q in range(1, NQ - 1):
        xw_ref[1:1 + D, 1:1 + H, q:q + 1, :] = (
            xb[:, :, 2 * q - 1:2 * q + 3, :].reshape(D, H, 1, KW))
    xw_ref[1:1 + D, 1:1 + H, 0:1, C:KW] = xb[:, :, 0:3, :].reshape(D, H, 1, 3 * C)
    xw_ref[1:1 + D, 1:1 + H, NQ - 1:NQ, 0:3 * C] = (
        xb[:, :, W - 3:W, :].reshape(D, H, 1, 3 * C))

    def conv(src_ref, w_ref):
        acc = jnp.zeros((M, NC), jnp.float32)
        for t in range(9):
            kd, kh = t // 3, t % 3
            lhs = src_ref[kd:kd + D, kh:kh + H, :, :].reshape(M, KW)
            acc = acc + jnp.dot(lhs, w_ref[t],
                                preferred_element_type=jnp.float32)
        return acc

    y = _leaky(conv(xw_ref, w1_ref) + t1_ref[...])
    yb = y.astype(bf16).reshape(D, H, NQ, 2, C)
    yw_ref[1:1 + D, 1:1 + H, :, C:3 * C] = yb.reshape(D, H, NQ, NC)
    yw_ref[1:1 + D, 1:1 + H, 0:NQ - 1, 3 * C:KW] = yb[:, :, 1:NQ, 0, :]
    yw_ref[1:1 + D, 1:1 + H, 1:NQ, 0:C] = yb[:, :, 0:NQ - 1, 1, :]

    z = conv(yw_ref, w2_ref) + t2_ref[...] + x_ref[...].reshape(M, NC)
    o_ref[...] = _leaky(z).reshape(D, H, W, C)


def _build_call(N, D, H, W, C):
    NQ = W // 2
    KW, NC = 4 * C, 2 * C
    vol = pl.BlockSpec((None, D, H, W, C), lambda n: (n, 0, 0, 0, 0))
    wspec = pl.BlockSpec((9, KW, NC), lambda n: (0, 0, 0))
    tspec = pl.BlockSpec((1, NC), lambda n: (0, 0))
    return pl.pallas_call(
        _block_kernel,
        out_shape=jax.ShapeDtypeStruct((N, D, H, W, C), jnp.float32),
        grid=(N,),
        in_specs=[vol, wspec, tspec, wspec, tspec],
        out_specs=vol,
        scratch_shapes=[
            pltpu.VMEM((D + 2, H + 2, NQ, KW), jnp.bfloat16),
            pltpu.VMEM((D + 2, H + 2, NQ, KW), jnp.bfloat16),
        ],
        compiler_params=pltpu.CompilerParams(
            dimension_semantics=("parallel",),
            vmem_limit_bytes=52 * 1024 * 1024,
        ),
    )


def _fold_band(w, conv_b, gamma, beta, mean, var, C):
    s = gamma * jax.lax.rsqrt(var + _EPS)
    t = conv_b * s + beta - mean * s
    wt = jnp.transpose(w * s[:, None, None, None, None],
                       (2, 3, 4, 1, 0))
    band = jnp.zeros((3, 3, 4, C, 2, C), jnp.float32)
    for wq in range(2):
        for kw in range(3):
            band = band.at[:, :, wq + kw, :, wq, :].set(wt[:, :, kw])
    band = band.reshape(9, 4 * C, 2 * C).astype(jnp.bfloat16)
    tcol = jnp.concatenate([t, t]).reshape(1, 2 * C).astype(jnp.float32)
    return band, tcol


def kernel(x, w1, b1, gamma1, beta1, mean1, var1,
           w2, b2, gamma2, beta2, mean2, var2):
    xn = jnp.transpose(x, (0, 2, 3, 4, 1)).astype(jnp.float32)
    N, D, H, W, C = xn.shape
    band1, t1c = _fold_band(w1, b1, gamma1, beta1, mean1, var1, C)
    band2, t2c = _fold_band(w2, b2, gamma2, beta2, mean2, var2, C)
    out = _build_call(N, D, H, W, C)(xn, band1, t1c, band2, t2c)
    return jnp.transpose(out, (0, 4, 1, 2, 3))
